# baseline (speedup 1.0000x reference)
import functools

import jax
import jax.numpy as jnp
from jax import lax
from jax.experimental import pallas as pl
from jax.experimental.pallas import tpu as pltpu
from jax.experimental.pallas import tpu_sc as plsc

BATCH = 16384
SEQ = 200
D = 64
B = BATCH * SEQ

_INFO = plsc.get_sparse_core_info()
NC = _INFO.num_cores
NS = _INFO.num_subcores
NW = NC * NS
B_PER_W = B // NW

K = 8
GSZ = K * 128
N_GROUPS = B_PER_W // GSZ


def _make_gather():
    mesh = plsc.VectorSubcoreMesh(core_axis_name="c", subcore_axis_name="s")

    @functools.partial(
        pl.kernel,
        mesh=mesh,
        out_type=jax.ShapeDtypeStruct((B, D), jnp.float32),
        scratch_types=[
            pltpu.VMEM((K, 128), jnp.int32),
            pltpu.VMEM((GSZ, D), jnp.float32),
            pltpu.SemaphoreType.DMA,
        ],
        compiler_params=pltpu.CompilerParams(use_tc_tiling_on_sc=False),
    )
    def gather_kernel(x_hbm, table_hbm, out_hbm, idx_v, rows_v, sem):
        wid = lax.axis_index("s") * NC + lax.axis_index("c")
        row0 = wid * (B_PER_W // 128)

        def group(g, carry):
            r = row0 + g * K
            pltpu.sync_copy(x_hbm.at[pl.ds(r, K)], idx_v)
            copies = [
                pltpu.async_copy(
                    table_hbm.at[idx_v.at[j]],
                    rows_v.at[pl.ds(j * 128, 128)],
                    sem,
                )
                for j in range(K)
            ]
            for c in copies:
                c.wait()
            pltpu.sync_copy(rows_v, out_hbm.at[pl.ds(r * 128, GSZ)])
            return carry

        lax.fori_loop(0, N_GROUPS, group, 0)

    return gather_kernel


_gather = _make_gather()


def kernel(x, table):
    x2d = x.reshape(B // 128, 128)
    out = _gather(x2d, table)
    return out.reshape(BATCH, SEQ, D)


# speedup vs baseline: 1.0274x; 1.0274x over previous
import functools

import jax
import jax.numpy as jnp
from jax import lax
from jax.experimental import pallas as pl
from jax.experimental.pallas import tpu as pltpu
from jax.experimental.pallas import tpu_sc as plsc

BATCH = 16384
SEQ = 200
D = 64
B = BATCH * SEQ

_INFO = plsc.get_sparse_core_info()
NC = _INFO.num_cores
NS = _INFO.num_subcores
NW = NC * NS
B_PER_W = B // NW

K = 4
GSZ = K * 128
N_GROUPS = B_PER_W // GSZ
NBUF = 3


def _make_gather():
    mesh = plsc.VectorSubcoreMesh(core_axis_name="c", subcore_axis_name="s")

    @functools.partial(
        pl.kernel,
        mesh=mesh,
        out_type=jax.ShapeDtypeStruct((B, D), jnp.float32),
        scratch_types=[
            [pltpu.VMEM((K, 128), jnp.int32) for _ in range(NBUF)],
            [pltpu.VMEM((GSZ, D), jnp.float32) for _ in range(NBUF)],
            [pltpu.SemaphoreType.DMA for _ in range(NBUF)],
            [pltpu.SemaphoreType.DMA for _ in range(NBUF)],
        ],
        compiler_params=pltpu.CompilerParams(use_tc_tiling_on_sc=False),
    )
    def gather_kernel(x_hbm, table_hbm, out_hbm, idx_v, rows_v, sem_g, sem_w):
        wid = lax.axis_index("s") * NC + lax.axis_index("c")
        row0 = wid * (B_PER_W // 128)

        def fire_gathers(g, c):
            pltpu.sync_copy(x_hbm.at[pl.ds(row0 + g * K, K)], idx_v[c])
            for j in range(K):
                pltpu.async_copy(
                    table_hbm.at[idx_v[c].at[j]],
                    rows_v[c].at[pl.ds(j * 128, 128)],
                    sem_g[c],
                )

        def wait_gathers(c):
            pltpu.make_async_copy(
                table_hbm.at[pl.ds(0, GSZ)], rows_v[c], sem_g[c]
            ).wait()

        def fire_writeback(g, c):
            pltpu.async_copy(
                rows_v[c], out_hbm.at[pl.ds((row0 + g * K) * 128, GSZ)], sem_w[c]
            )

        def wait_writeback(c):
            pltpu.make_async_copy(
                rows_v[c], out_hbm.at[pl.ds(0, GSZ)], sem_w[c]
            ).wait()

        fire_gathers(0, 0)
        fire_gathers(1, 1)
        wait_gathers(0)
        fire_writeback(0, 0)
        fire_gathers(2, 2)
        wait_gathers(1)
        fire_writeback(1, 1)

        TRIPS = (N_GROUPS - NBUF) // NBUF

        def trip(t, carry):
            g0 = NBUF + t * NBUF
            for u in range(NBUF):
                c = u
                wait_writeback(c)
                fire_gathers(g0 + u, c)
                wait_gathers((c + NBUF - 1) % NBUF)
                fire_writeback(g0 + u - 1, (c + NBUF - 1) % NBUF)
            return carry

        lax.fori_loop(0, TRIPS, trip, 0)

        g = NBUF + TRIPS * NBUF
        wait_writeback(0)
        fire_gathers(g, 0)
        wait_gathers(2)
        fire_writeback(g - 1, 2)
        wait_writeback(1)
        fire_gathers(g + 1, 1)
        wait_gathers(0)
        fire_writeback(g, 0)
        wait_gathers(1)
        fire_writeback(g + 1, 1)
        wait_writeback(0)
        wait_writeback(1)
        wait_writeback(2)

    return gather_kernel


_gather = _make_gather()


def kernel(x, table):
    x2d = x.reshape(B // 128, 128)
    out = _gather(x2d, table)
    return out.reshape(BATCH, SEQ, D)


# speedup vs baseline: 1.3310x; 1.2954x over previous
import functools

import jax
import jax.numpy as jnp
from jax import lax
from jax.experimental import pallas as pl
from jax.experimental.pallas import tpu as pltpu
from jax.experimental.pallas import tpu_sc as plsc

BATCH = 16384
SEQ = 200
D = 64
B = BATCH * SEQ

_INFO = plsc.get_sparse_core_info()
NC = _INFO.num_cores
NS = _INFO.num_subcores
NW = NC * NS
B_PER_W = B // NW
GSZ = 320
N_GROUPS = B_PER_W // GSZ
CHUNKS = ((0, 128), (128, 128), (256, 64))


def _make_gather():
    mesh = plsc.VectorSubcoreMesh(core_axis_name="c", subcore_axis_name="s")

    @functools.partial(
        pl.kernel,
        mesh=mesh,
        out_type=jax.ShapeDtypeStruct((B, 128), jnp.float32),
        scratch_types=[
            [pltpu.VMEM((GSZ,), jnp.int32) for _ in range(2)],
            [pltpu.VMEM((GSZ, 128), jnp.float32) for _ in range(2)],
            [pltpu.SemaphoreType.DMA for _ in range(2)],
            [pltpu.SemaphoreType.DMA for _ in range(2)],
        ],
        compiler_params=pltpu.CompilerParams(use_tc_tiling_on_sc=False),
    )
    def gather_kernel(x_hbm, tab_hbm, out_hbm, idx_v, rows_v, sem_g, sem_w):
        wid = lax.axis_index("s") * NC + lax.axis_index("c")
        r00 = wid * B_PER_W

        def fire_gather(g, c):
            r0 = r00 + g * GSZ
            pltpu.sync_copy(x_hbm.at[pl.ds(r0, GSZ)], idx_v[c])
            for off, n in CHUNKS:
                pltpu.async_copy(
                    tab_hbm.at[idx_v[c].at[pl.ds(off, n)]],
                    rows_v[c].at[pl.ds(off, n)],
                    sem_g[c],
                )

        def wait_gather(c):
            pltpu.make_async_copy(
                tab_hbm.at[pl.ds(0, GSZ)], rows_v[c], sem_g[c]
            ).wait()

        def fire_wb(g, c):
            r0 = r00 + g * GSZ
            pltpu.async_copy(rows_v[c], out_hbm.at[pl.ds(r0, GSZ)], sem_w[c])

        def wait_wb(c):
            pltpu.make_async_copy(
                rows_v[c], out_hbm.at[pl.ds(0, GSZ)], sem_w[c]
            ).wait()

        def stage(g, c, fire_next, wait_out):
            if wait_out:
                wait_wb(1 - c)
            if fire_next:
                fire_gather(g + 1, 1 - c)
            wait_gather(c)
            fire_wb(g, c)

        fire_gather(0, 0)
        stage(0, 0, True, False)
        stage(1, 1, True, True)

        TRIPS = (N_GROUPS - 4) // 2

        def trip(t, carry):
            g = 2 + 2 * t
            stage(g, 0, True, True)
            stage(g + 1, 1, True, True)
            return carry

        lax.fori_loop(0, TRIPS, trip, 0)

        stage(N_GROUPS - 2, 0, True, True)
        stage(N_GROUPS - 1, 1, False, True)
        wait_wb(1)

    return gather_kernel


_gather = _make_gather()


def kernel(x, table):
    x1d = x.reshape(-1)
    table_pad = jnp.pad(table, ((0, 0), (0, 128 - D)))
    out2 = _gather(x1d, table_pad)
    return out2[:, :D].reshape(BATCH, SEQ, D)


# speedup vs baseline: 1.3319x; 1.0007x over previous
import functools

import jax
import jax.numpy as jnp
from jax import lax
from jax.experimental import pallas as pl
from jax.experimental.pallas import tpu as pltpu
from jax.experimental.pallas import tpu_sc as plsc

BATCH = 16384
SEQ = 200
D = 64
B = BATCH * SEQ

_INFO = plsc.get_sparse_core_info()
NC = _INFO.num_cores
NS = _INFO.num_subcores
NW = NC * NS
B_PER_W = B // NW
GSZ = 320
N_GROUPS = B_PER_W // GSZ
CHUNKS = ((0, 128), (128, 128), (256, 64))


def _make_gather():
    mesh = plsc.VectorSubcoreMesh(core_axis_name="c", subcore_axis_name="s")

    @functools.partial(
        pl.kernel,
        mesh=mesh,
        out_type=jax.ShapeDtypeStruct((B, 128), jnp.float32),
        scratch_types=[
            [pltpu.VMEM((GSZ,), jnp.int32) for _ in range(3)],
            [pltpu.VMEM((GSZ, 128), jnp.float32) for _ in range(3)],
            [pltpu.SemaphoreType.DMA for _ in range(3)],
            [pltpu.SemaphoreType.DMA for _ in range(3)],
        ],
        compiler_params=pltpu.CompilerParams(use_tc_tiling_on_sc=False),
    )
    def gather_kernel(x_hbm, tab_hbm, out_hbm, idx_v, rows_v, sem_g, sem_w):
        wid = lax.axis_index("s") * NC + lax.axis_index("c")
        r00 = wid * B_PER_W

        def fire_gather(g, c):
            r0 = r00 + g * GSZ
            pltpu.sync_copy(x_hbm.at[pl.ds(r0, GSZ)], idx_v[c])
            for off, n in CHUNKS:
                pltpu.async_copy(
                    tab_hbm.at[idx_v[c].at[pl.ds(off, n)]],
                    rows_v[c].at[pl.ds(off, n)],
                    sem_g[c],
                )

        def wait_gather(c):
            pltpu.make_async_copy(
                tab_hbm.at[pl.ds(0, GSZ)], rows_v[c], sem_g[c]
            ).wait()

        def fire_wb(g, c):
            r0 = r00 + g * GSZ
            pltpu.async_copy(rows_v[c], out_hbm.at[pl.ds(r0, GSZ)], sem_w[c])

        def wait_wb(c):
            pltpu.make_async_copy(
                rows_v[c], out_hbm.at[pl.ds(0, GSZ)], sem_w[c]
            ).wait()

        fire_gather(0, 0)
        fire_gather(1, 1)
        wait_gather(0)
        fire_wb(0, 0)
        fire_gather(2, 2)
        wait_gather(1)
        fire_wb(1, 1)

        TRIPS = (N_GROUPS - 5) // 3

        def body(g, c):
            wait_wb(c)
            fire_gather(g, c)
            wait_gather((c + 2) % 3)
            fire_wb(g - 1, (c + 2) % 3)

        def trip(t, carry):
            g = 3 + 3 * t
            body(g, 0)
            body(g + 1, 1)
            body(g + 2, 2)
            return carry

        lax.fori_loop(0, TRIPS, trip, 0)

        body(N_GROUPS - 2, 0)
        body(N_GROUPS - 1, 1)
        wait_gather(1)
        fire_wb(N_GROUPS - 1, 1)
        wait_wb(0)
        wait_wb(1)
        wait_wb(2)

    return gather_kernel


_gather = _make_gather()


def kernel(x, table):
    x1d = x.reshape(-1)
    table_pad = jnp.pad(table, ((0, 0), (0, 128 - D)))
    out2 = _gather(x1d, table_pad)
    return out2[:, :D].reshape(BATCH, SEQ, D)
